# Initial kernel scaffold; baseline (speedup 1.0000x reference)
import jax
import jax.numpy as jnp
from jax.experimental import pallas as pl


def kernel(x_t, x_t1, edge_index, edge_attr):
    raise NotImplementedError("write your pallas kernel here")


# speedup vs baseline: 234.5202x; 234.5202x over previous
import functools

import jax
import jax.numpy as jnp
from jax import lax
from jax.experimental import pallas as pl
from jax.experimental.pallas import tpu as pltpu, tpu_sc as plsc

N_NODES = 100000
N_EDGES = 3200000
DELTA_T = 0.01
MU = 0.01

LANES = 16
N_TILES = 32
ROW = 128
N_ROWS = N_EDGES // ROW
SC_ROWS = 16
CHUNK = SC_ROWS * ROW
BIG_TILES = 21
TAIL_ROWS = 8

NODE_SLICE = 6256
LAST_SLICE = N_NODES - 15 * NODE_SLICE
CB = 2000


def _edge_pass(combine, src_hbm, dstR_hbm, w_hbm, *refs):
    if combine:
        (g0_hbm, g1_hbm, s_out, p0_out, p1_out, u_v, zb_v, cb_v, src_v,
         dst2_v, w_v, msg_v, acc, sem) = refs
    else:
        (g0_hbm, p0_out, p1_out, u_v, zb_v, cb_v, src_v, dst2_v,
         w_v, msg_v, acc, sem) = refs

    c = lax.axis_index("c")
    s = lax.axis_index("s")
    wid = c * 16 + s

    pltpu.sync_copy(g0_hbm, u_v)
    if combine:
        def add_chunk(i, _):
            pltpu.sync_copy(g1_hbm.at[pl.ds(i * CB, CB)], cb_v)

            def add16(k, _):
                off = i * CB + k * LANES
                u_v[pl.ds(off, LANES)] = (
                    u_v[pl.ds(off, LANES)] + cb_v[pl.ds(k * LANES, LANES)])
                return 0

            lax.fori_loop(0, CB // LANES, add16, 0)
            return 0

        lax.fori_loop(0, N_NODES // CB, add_chunk, 0)

    def zero16(i, _):
        zb_v[pl.ds(i * LANES, LANES)] = jnp.zeros((LANES,), jnp.float32)
        return 0

    lax.fori_loop(0, NODE_SLICE // LANES, zero16, 0)

    @pl.when(s < 15)
    def _():
        pltpu.sync_copy(zb_v, acc.at[pl.ds(s * NODE_SLICE, NODE_SLICE)])

    @pl.when(s == 15)
    def _():
        pltpu.sync_copy(zb_v.at[pl.ds(0, LAST_SLICE)],
                        acc.at[pl.ds(15 * NODE_SLICE, LAST_SLICE)])

    plsc.subcore_barrier()

    is_big = wid < BIG_TILES
    r0 = 784 * wid - TAIL_ROWS * jnp.maximum(wid - BIG_TILES, 0)
    n_full = 48 + jnp.where(is_big, 1, 0)

    def compute_rows(nrows_static):
        for j in range(nrows_static):
            def lane_grp(k, _, j=j):
                off = j * ROW + k * LANES
                s_idx = src_v[pl.ds(off, LANES)]
                d_idx = dst2_v[j, pl.ds(k * LANES, LANES)]
                wv = w_v[pl.ds(off, LANES)]
                us = plsc.load_gather(u_v, [s_idx])
                ud = plsc.load_gather(u_v, [d_idx])
                msg_v[j, pl.ds(k * LANES, LANES)] = (us - ud) * wv
                return 0

            lax.fori_loop(0, ROW // LANES, lane_grp, 0)

    def scatter_rows(nrows_static):
        descs = [
            pltpu.async_copy(msg_v.at[j], acc.at[dst2_v.at[j]], sem, add=True)
            for j in range(nrows_static)
        ]
        for d in descs:
            d.wait()

    def superchunk(sc, _):
        r = r0 + sc * SC_ROWS
        e0 = r * ROW
        pltpu.sync_copy(src_hbm.at[pl.ds(e0, CHUNK)], src_v)
        pltpu.sync_copy(dstR_hbm.at[pl.ds(r, SC_ROWS)], dst2_v)
        pltpu.sync_copy(w_hbm.at[pl.ds(e0, CHUNK)], w_v)
        compute_rows(SC_ROWS)
        scatter_rows(SC_ROWS)
        return 0

    lax.fori_loop(0, n_full, superchunk, 0)

    @pl.when(jnp.logical_not(is_big))
    def _():
        r = r0 + 48 * SC_ROWS
        e0 = r * ROW
        n = TAIL_ROWS * ROW
        pltpu.sync_copy(src_hbm.at[pl.ds(e0, n)], src_v.at[pl.ds(0, n)])
        pltpu.sync_copy(dstR_hbm.at[pl.ds(r, TAIL_ROWS)],
                        dst2_v.at[pl.ds(0, TAIL_ROWS)])
        pltpu.sync_copy(w_hbm.at[pl.ds(e0, n)], w_v.at[pl.ds(0, n)])
        compute_rows(TAIL_ROWS)
        scatter_rows(TAIL_ROWS)

    plsc.subcore_barrier()

    def writeback(dst_hbm_ref, src_ref, bounce):
        @pl.when(s < 15)
        def _():
            if bounce:
                pltpu.sync_copy(src_ref.at[pl.ds(s * NODE_SLICE, NODE_SLICE)],
                                zb_v)
                pltpu.sync_copy(
                    zb_v, dst_hbm_ref.at[pl.ds(s * NODE_SLICE, NODE_SLICE)])
            else:
                pltpu.sync_copy(
                    src_ref.at[pl.ds(s * NODE_SLICE, NODE_SLICE)],
                    dst_hbm_ref.at[pl.ds(s * NODE_SLICE, NODE_SLICE)])

        @pl.when(s == 15)
        def _():
            if bounce:
                pltpu.sync_copy(acc.at[pl.ds(15 * NODE_SLICE, LAST_SLICE)],
                                zb_v.at[pl.ds(0, LAST_SLICE)])
                pltpu.sync_copy(
                    zb_v.at[pl.ds(0, LAST_SLICE)],
                    dst_hbm_ref.at[pl.ds(15 * NODE_SLICE, LAST_SLICE)])
            else:
                pltpu.sync_copy(
                    src_ref.at[pl.ds(15 * NODE_SLICE, LAST_SLICE)],
                    dst_hbm_ref.at[pl.ds(15 * NODE_SLICE, LAST_SLICE)])

    @pl.when(c == 0)
    def _():
        writeback(p0_out, acc, True)
        if combine:
            writeback(s_out, u_v, False)

    @pl.when(c == 1)
    def _():
        writeback(p1_out, acc, True)


def _make_pass(combine):
    mesh = plsc.VectorSubcoreMesh(core_axis_name="c", subcore_axis_name="s")
    node = jax.ShapeDtypeStruct((N_NODES,), jnp.float32)
    outs = (node, node, node) if combine else (node, node)
    scratches = [
        pltpu.VMEM((N_NODES,), jnp.float32),
        pltpu.VMEM((NODE_SLICE,), jnp.float32),
        pltpu.VMEM((CB,), jnp.float32),
        pltpu.VMEM((CHUNK,), jnp.int32),
        pltpu.VMEM((SC_ROWS, ROW), jnp.int32),
        pltpu.VMEM((CHUNK,), jnp.float32),
        pltpu.VMEM((SC_ROWS, ROW), jnp.float32),
        pltpu.VMEM_SHARED((N_NODES,), jnp.float32),
        pltpu.SemaphoreType.DMA,
    ]
    return pl.kernel(
        functools.partial(_edge_pass, combine),
        out_type=outs,
        mesh=mesh,
        scratch_types=scratches,
        compiler_params=pltpu.CompilerParams(needs_layout_passes=False),
        name="burger_pass2" if combine else "burger_pass1",
    )


def _residual_body(ut_ref, ut1_ref, s1_ref, p0_ref, p1_ref, o_ref):
    ut = ut_ref[...]
    ut1 = ut1_ref[...]
    s1 = s1_ref[...]
    s2 = p0_ref[...] + p1_ref[...]
    o_ref[...] = (ut - ut1) / DELTA_T + s1 * ut1 - MU * s2


def kernel(x_t, x_t1, edge_index, edge_attr):
    u_t = x_t[:, 0]
    u_t1 = x_t1[:, 0]
    src = edge_index[0]
    dstR = edge_index[1].reshape(N_ROWS, ROW)
    w = edge_attr[:, 0]

    pass1 = _make_pass(False)
    pass2 = _make_pass(True)

    p0, p1 = pass1(src, dstR, w, u_t1)
    s1, q0, q1 = pass2(src, dstR, w, p0, p1)

    shape2d = (8, N_NODES // 8)
    loss = pl.pallas_call(
        _residual_body,
        out_shape=jax.ShapeDtypeStruct(shape2d, jnp.float32),
    )(u_t.reshape(shape2d), u_t1.reshape(shape2d), s1.reshape(shape2d),
      q0.reshape(shape2d), q1.reshape(shape2d))
    return loss.reshape(N_NODES)
